# Initial kernel scaffold; baseline (speedup 1.0000x reference)
import jax
import jax.numpy as jnp
from jax.experimental import pallas as pl


def kernel(gene_feat, drug_feat, edge_index_gene_gene, edge_index_drug_drug, edge_index_target_drug, edge_index_drug_target, W_gg, b_gg, W_dd, b_dd, W_td, b_td, W_dt, b_dt):
    raise NotImplementedError("write your pallas kernel here")


# speedup vs baseline: 8.8558x; 8.8558x over previous
import functools

import jax
import jax.numpy as jnp
from jax import lax
from jax.experimental import pallas as pl
from jax.experimental.pallas import tpu as pltpu
from jax.experimental.pallas import tpu_sc as plsc

N = 10000
D = 128
E = 320000
L = 16
NPAD = 10240
NS = 16
CH = 128
RPT = 160
E2 = NS * RPT * CH
STR = NPAD // NS
NH = 5120
NR = NH + 256
NRS = NR // NS

_sc_mesh = plsc.VectorSubcoreMesh(core_axis_name="c", subcore_axis_name="s")


@functools.partial(
    pl.kernel,
    out_type=jax.ShapeDtypeStruct((4, 2, NR, D), jnp.float32),
    mesh=_sc_mesh,
    scratch_types=[
        pltpu.VMEM((RPT, CH), jnp.int32),
        pltpu.VMEM((RPT, CH), jnp.int32),
        pltpu.VMEM((CH, D), jnp.float32),
        pltpu.VMEM_SHARED((NR, D), jnp.float32),
        pltpu.SemaphoreType.DMA,
    ],
)
def _scatter_kernel(ytab, srcs, dsts2, zstr, out, idxs, idxd, rows,
                    acc, sem):
    c = lax.axis_index("c")
    s = lax.axis_index("s")

    for li in range(2):
        rel = 2 * c + li
        pltpu.sync_copy(srcs.at[rel, pl.ds(s * RPT, RPT)], idxs)

        for p in range(2):
            pltpu.sync_copy(dsts2.at[rel, p, pl.ds(s * RPT, RPT)], idxd)
            pltpu.sync_copy(zstr, acc.at[pl.ds(s * NRS, NRS)])
            plsc.subcore_barrier()

            @pl.loop(0, RPT)
            def _chunk(r):
                pltpu.async_copy(ytab.at[idxs.at[r]], rows, sem).wait()
                pltpu.sync_copy(rows, acc.at[idxd.at[r]], add=True)
            plsc.subcore_barrier()
            pltpu.sync_copy(acc.at[pl.ds(s * NRS, NRS)],
                            out.at[rel, p, pl.ds(s * NRS, NRS)])


def _dinv_body(degs_ref, out_ref):
    d = degs_ref[...]
    row = lax.broadcasted_iota(jnp.int32, d.shape, 0)
    deff = jnp.where(row < 2, d + 1.0, jnp.maximum(d, 1.0))
    out_ref[...] = lax.rsqrt(deff)


def _prep_body(x_ref, w_ref, dinv_ref, y_ref):
    xw = jnp.dot(x_ref[0], w_ref[0], preferred_element_type=jnp.float32)
    y_ref[0] = xw * dinv_ref[0]


def _final_body(acc_ref, y01_ref, dinvd_ref, b_ref, drug_ref, gene_ref):
    outs = []
    for r in range(4):
        a = acc_ref[r]
        if r < 2:
            a = a + y01_ref[r]
        pre = a * dinvd_ref[r] + b_ref[pl.ds(r, 1), :]
        v = jnp.maximum(pre, 0.0)
        ssq = jnp.sum(v * v, axis=1, keepdims=True)
        outs.append(v * (1.0 / jnp.maximum(jnp.sqrt(ssq), 1e-12)))
    drug_ref[...] = outs[1] + outs[2]
    gene_ref[...] = outs[0] + outs[3]


_RB = 2000
_PADE = E2 - E


def _pad_idx(v, fill):
    return jnp.concatenate([v, fill]).reshape(E2 // CH, CH)


def kernel(gene_feat, drug_feat, edge_index_gene_gene, edge_index_drug_drug,
           edge_index_target_drug, edge_index_drug_target,
           W_gg, b_gg, W_dd, b_dd, W_td, b_td, W_dt, b_dt):
    e = [edge_index_gene_gene, edge_index_drug_drug,
         edge_index_target_drug, edge_index_drug_target]
    pad_pos = jnp.arange(_PADE, dtype=jnp.int32)
    pad_junk = N + (pad_pos % CH)
    pad_src = pad_pos % 1024
    lists6 = jnp.stack([
        _pad_idx(e[0][1], pad_junk), _pad_idx(e[1][1], pad_junk),
        _pad_idx(e[2][0], pad_junk), _pad_idx(e[2][1], pad_junk),
        _pad_idx(e[3][0], pad_junk), _pad_idx(e[3][1], pad_junk)])
    srcs = jnp.stack([_pad_idx(e[r][0], pad_src) + r * N for r in range(4)])
    dsts = jnp.stack([_pad_idx(e[r][1], pad_junk) for r in range(4)])
    lane = jnp.arange(CH, dtype=jnp.int32)[None, None, None, :]
    dsts2 = jnp.stack([dsts - p * NH for p in range(2)], axis=1)
    dsts2 = jnp.where((dsts2 >= 0) & (dsts2 < NH), dsts2, NH + lane)

    zstr = jnp.zeros((NRS, D), jnp.float32)

    degs = jnp.stack([jnp.zeros((NPAD,), jnp.float32).at[lists6[j].reshape(-1)
                                                         ].add(1.0)
                      for j in range(6)])

    dinv6 = pl.pallas_call(
        _dinv_body,
        out_shape=jax.ShapeDtypeStruct((6, NPAD), jnp.float32),
    )(degs)[:, :N]

    dinv_src = jnp.stack([dinv6[0], dinv6[1], dinv6[2], dinv6[4]])[..., None]
    dinv_dst = jnp.stack([dinv6[0], dinv6[1], dinv6[3], dinv6[5]])[..., None]

    X = jnp.stack([gene_feat, drug_feat, gene_feat, drug_feat])
    W4 = jnp.stack([W_gg, W_dd, W_td, W_dt])
    B4 = jnp.stack([b_gg, b_dd, b_td, b_dt])

    y = pl.pallas_call(
        _prep_body,
        grid=(4, N // _RB),
        in_specs=[
            pl.BlockSpec((1, _RB, D), lambda r, i: (r, i, 0)),
            pl.BlockSpec((1, D, D), lambda r, i: (r, 0, 0)),
            pl.BlockSpec((1, _RB, 1), lambda r, i: (r, i, 0)),
        ],
        out_specs=pl.BlockSpec((1, _RB, D), lambda r, i: (r, i, 0)),
        out_shape=jax.ShapeDtypeStruct((4, N, D), jnp.float32),
    )(X, W4, dinv_src)

    accO = _scatter_kernel(y.reshape(4 * N, D), srcs, dsts2, zstr)
    acc = jnp.concatenate([accO[:, 0, :NH], accO[:, 1, :N - NH]], axis=1)

    drug_out, gene_out = pl.pallas_call(
        _final_body,
        grid=(N // _RB,),
        in_specs=[
            pl.BlockSpec((4, _RB, D), lambda i: (0, i, 0)),
            pl.BlockSpec((2, _RB, D), lambda i: (0, i, 0)),
            pl.BlockSpec((4, _RB, 1), lambda i: (0, i, 0)),
            pl.BlockSpec((4, D), lambda i: (0, 0)),
        ],
        out_specs=[
            pl.BlockSpec((_RB, D), lambda i: (i, 0)),
            pl.BlockSpec((_RB, D), lambda i: (i, 0)),
        ],
        out_shape=[
            jax.ShapeDtypeStruct((N, D), jnp.float32),
            jax.ShapeDtypeStruct((N, D), jnp.float32),
        ],
    )(acc, y, dinv_dst, B4)
    return drug_out, gene_out


# speedup vs baseline: 10.1350x; 1.1444x over previous
import functools

import jax
import jax.numpy as jnp
from jax import lax
from jax.experimental import pallas as pl
from jax.experimental.pallas import tpu as pltpu
from jax.experimental.pallas import tpu_sc as plsc

N = 10000
D = 128
E = 320000
L = 16
NPAD = 10240
NS = 16
CH = 128
RPT = 160
E2 = NS * RPT * CH
STR = NPAD // NS
NH = 5120
NR = NH + 256
NRS = NR // NS

_sc_mesh = plsc.VectorSubcoreMesh(core_axis_name="c", subcore_axis_name="s")


@functools.partial(
    pl.kernel,
    out_type=jax.ShapeDtypeStruct((4, 2, NR, D), jnp.float32),
    mesh=_sc_mesh,
    scratch_types=[
        pltpu.VMEM((RPT, CH), jnp.int32),
        pltpu.VMEM((RPT, CH), jnp.int32),
        pltpu.VMEM((2, CH, D), jnp.float32),
        pltpu.VMEM_SHARED((NR, D), jnp.float32),
        pltpu.SemaphoreType.DMA,
        pltpu.SemaphoreType.DMA,
    ],
)
def _scatter_kernel(ytab, srcs, dsts2, zstr, out, idxs, idxd, rows,
                    acc, sem0, sem1):
    c = lax.axis_index("c")
    s = lax.axis_index("s")

    for li in range(2):
        rel = 2 * c + li
        pltpu.sync_copy(srcs.at[rel, pl.ds(s * RPT, RPT)], idxs)

        for p in range(2):
            pltpu.sync_copy(dsts2.at[rel, p, pl.ds(s * RPT, RPT)], idxd)
            pltpu.sync_copy(zstr, acc.at[pl.ds(s * NRS, NRS)])
            plsc.subcore_barrier()

            pltpu.async_copy(ytab.at[idxs.at[0]], rows.at[0], sem0)

            @pl.loop(0, RPT, step=2)
            def _chunk(r):
                pltpu.make_async_copy(ytab.at[idxs.at[r]], rows.at[0],
                                      sem0).wait()

                @pl.when(r + 1 < RPT)
                def _fire1():
                    pltpu.async_copy(ytab.at[idxs.at[r + 1]], rows.at[1],
                                     sem1)

                pltpu.sync_copy(rows.at[0], acc.at[idxd.at[r]], add=True)

                @pl.when(r + 1 < RPT)
                def _second():
                    pltpu.make_async_copy(ytab.at[idxs.at[r + 1]],
                                          rows.at[1], sem1).wait()

                    @pl.when(r + 2 < RPT)
                    def _fire0():
                        pltpu.async_copy(ytab.at[idxs.at[r + 2]],
                                         rows.at[0], sem0)

                    pltpu.sync_copy(rows.at[1], acc.at[idxd.at[r + 1]],
                                    add=True)
            plsc.subcore_barrier()
            pltpu.sync_copy(acc.at[pl.ds(s * NRS, NRS)],
                            out.at[rel, p, pl.ds(s * NRS, NRS)])


def _dinv_body(degs_ref, out_ref):
    d = degs_ref[...]
    row = lax.broadcasted_iota(jnp.int32, d.shape, 0)
    deff = jnp.where(row < 2, d + 1.0, jnp.maximum(d, 1.0))
    out_ref[...] = lax.rsqrt(deff)


def _prep_body(x_ref, w_ref, dinv_ref, y_ref):
    xw = jnp.dot(x_ref[0], w_ref[0], preferred_element_type=jnp.float32)
    y_ref[0] = xw * dinv_ref[0]


def _final_body(acc_ref, y01_ref, dinvd_ref, b_ref, drug_ref, gene_ref):
    outs = []
    for r in range(4):
        a = acc_ref[r]
        if r < 2:
            a = a + y01_ref[r]
        pre = a * dinvd_ref[r] + b_ref[pl.ds(r, 1), :]
        v = jnp.maximum(pre, 0.0)
        ssq = jnp.sum(v * v, axis=1, keepdims=True)
        outs.append(v * (1.0 / jnp.maximum(jnp.sqrt(ssq), 1e-12)))
    drug_ref[...] = outs[1] + outs[2]
    gene_ref[...] = outs[0] + outs[3]


_RB = 2000
_PADE = E2 - E


def _pad_idx(v, fill):
    return jnp.concatenate([v, fill]).reshape(E2 // CH, CH)


def kernel(gene_feat, drug_feat, edge_index_gene_gene, edge_index_drug_drug,
           edge_index_target_drug, edge_index_drug_target,
           W_gg, b_gg, W_dd, b_dd, W_td, b_td, W_dt, b_dt):
    e = [edge_index_gene_gene, edge_index_drug_drug,
         edge_index_target_drug, edge_index_drug_target]
    pad_pos = jnp.arange(_PADE, dtype=jnp.int32)
    pad_junk = N + (pad_pos % CH)
    pad_src = pad_pos % 1024
    lists6 = jnp.stack([
        _pad_idx(e[0][1], pad_junk), _pad_idx(e[1][1], pad_junk),
        _pad_idx(e[2][0], pad_junk), _pad_idx(e[2][1], pad_junk),
        _pad_idx(e[3][0], pad_junk), _pad_idx(e[3][1], pad_junk)])
    srcs = jnp.stack([_pad_idx(e[r][0], pad_src) + r * N for r in range(4)])
    dsts = jnp.stack([_pad_idx(e[r][1], pad_junk) for r in range(4)])
    lane = jnp.arange(CH, dtype=jnp.int32)[None, None, None, :]
    dsts2 = jnp.stack([dsts - p * NH for p in range(2)], axis=1)
    dsts2 = jnp.where((dsts2 >= 0) & (dsts2 < NH), dsts2, NH + lane)

    zstr = jnp.zeros((NRS, D), jnp.float32)

    degs = jnp.stack([jnp.zeros((NPAD,), jnp.float32).at[lists6[j].reshape(-1)
                                                         ].add(1.0)
                      for j in range(6)])

    dinv6 = pl.pallas_call(
        _dinv_body,
        out_shape=jax.ShapeDtypeStruct((6, NPAD), jnp.float32),
    )(degs)[:, :N]

    dinv_src = jnp.stack([dinv6[0], dinv6[1], dinv6[2], dinv6[4]])[..., None]
    dinv_dst = jnp.stack([dinv6[0], dinv6[1], dinv6[3], dinv6[5]])[..., None]

    X = jnp.stack([gene_feat, drug_feat, gene_feat, drug_feat])
    W4 = jnp.stack([W_gg, W_dd, W_td, W_dt])
    B4 = jnp.stack([b_gg, b_dd, b_td, b_dt])

    y = pl.pallas_call(
        _prep_body,
        grid=(4, N // _RB),
        in_specs=[
            pl.BlockSpec((1, _RB, D), lambda r, i: (r, i, 0)),
            pl.BlockSpec((1, D, D), lambda r, i: (r, 0, 0)),
            pl.BlockSpec((1, _RB, 1), lambda r, i: (r, i, 0)),
        ],
        out_specs=pl.BlockSpec((1, _RB, D), lambda r, i: (r, i, 0)),
        out_shape=jax.ShapeDtypeStruct((4, N, D), jnp.float32),
    )(X, W4, dinv_src)

    accO = _scatter_kernel(y.reshape(4 * N, D), srcs, dsts2, zstr)
    acc = jnp.concatenate([accO[:, 0, :NH], accO[:, 1, :N - NH]], axis=1)

    drug_out, gene_out = pl.pallas_call(
        _final_body,
        grid=(N // _RB,),
        in_specs=[
            pl.BlockSpec((4, _RB, D), lambda i: (0, i, 0)),
            pl.BlockSpec((2, _RB, D), lambda i: (0, i, 0)),
            pl.BlockSpec((4, _RB, 1), lambda i: (0, i, 0)),
            pl.BlockSpec((4, D), lambda i: (0, 0)),
        ],
        out_specs=[
            pl.BlockSpec((_RB, D), lambda i: (i, 0)),
            pl.BlockSpec((_RB, D), lambda i: (i, 0)),
        ],
        out_shape=[
            jax.ShapeDtypeStruct((N, D), jnp.float32),
            jax.ShapeDtypeStruct((N, D), jnp.float32),
        ],
    )(acc, y, dinv_dst, B4)
    return drug_out, gene_out
